# baseline (speedup 1.0000x reference)
import functools

import jax
import jax.numpy as jnp
from jax import lax
from jax.experimental import pallas as pl
from jax.experimental.pallas import tpu as pltpu
from jax.experimental.pallas import tpu_sc as plsc

K = 16
E = 64
H = 4
DH = E // H
DPAD = 128

_HI = jax.lax.Precision.HIGHEST


def _elu(x):
    return jnp.where(x > 0, x, jnp.exp(x) - 1.0)


def _embed_body(tok_ref, llm_ref, xyz_ref, emb_ref, g_ref, b_ref, w1_ref,
                b1_ref, w2_ref, b2_ref, out_ref):
    x = llm_ref[...]
    d = x.shape[1]
    m = jnp.sum(x, axis=1, keepdims=True) * (1.0 / d)
    v = jnp.sum(x * x, axis=1, keepdims=True) * (1.0 / d) - m * m
    h = (x - m) * (lax.rsqrt(v + 1e-5) * g_ref[...]) + b_ref[...]
    h = _elu(jnp.dot(h, w1_ref[...], preferred_element_type=jnp.float32)
             + b1_ref[...])
    h = _elu(jnp.dot(h, w2_ref[...], preferred_element_type=jnp.float32)
             + b2_ref[...])
    tok = tok_ref[...]
    blk = tok.shape[0]
    oh = (tok == lax.broadcasted_iota(jnp.int32, (blk, 32), 1)
          ).astype(jnp.float32)
    ft = jnp.dot(oh, emb_ref[...], precision=_HI,
                 preferred_element_type=jnp.float32)
    xyz = xyz_ref[...]
    pad = jnp.zeros((blk, DPAD - E - 9), jnp.float32)
    out_ref[...] = jnp.concatenate([ft, h, xyz, xyz, xyz, pad], axis=1)


def _embed(tok2d, llm, xyz, emb_pad, ln_g, ln_b, w1, b1, w2, b2):
    n, d = llm.shape
    blk = 1000
    assert n % blk == 0
    return pl.pallas_call(
        _embed_body,
        grid=(n // blk,),
        in_specs=[
            pl.BlockSpec((blk, 1), lambda i: (i, 0)),
            pl.BlockSpec((blk, d), lambda i: (i, 0)),
            pl.BlockSpec((blk, 3), lambda i: (i, 0)),
            pl.BlockSpec((32, 32), lambda i: (0, 0)),
            pl.BlockSpec((1, d), lambda i: (0, 0)),
            pl.BlockSpec((1, d), lambda i: (0, 0)),
            pl.BlockSpec((d, E), lambda i: (0, 0)),
            pl.BlockSpec((1, E), lambda i: (0, 0)),
            pl.BlockSpec((E, 32), lambda i: (0, 0)),
            pl.BlockSpec((1, 32), lambda i: (0, 0)),
        ],
        out_specs=pl.BlockSpec((blk, DPAD), lambda i: (i, 0)),
        out_shape=jax.ShapeDtypeStruct((n, DPAD), jnp.float32),
    )(tok2d, llm, xyz, emb_pad, ln_g, ln_b, w1, b1, w2, b2)


def _sc_gather(table, idx):
    bidx = idx.shape[0]
    info = plsc.get_sparse_core_info()
    nw = info.num_cores * info.num_subcores
    per_w = bidx // nw
    assert per_w * nw == bidx
    ch = 440
    while per_w % ch or ch % 8:
        ch -= 8
    nchunk = per_w // ch
    idx2 = idx.reshape(nw, per_w)
    mesh = plsc.VectorSubcoreMesh(core_axis_name="c", subcore_axis_name="s")

    @functools.partial(
        pl.kernel, mesh=mesh,
        out_type=jax.ShapeDtypeStruct((nw, nchunk, ch, DPAD), jnp.float32),
        scratch_types=[
            pltpu.VMEM((per_w,), jnp.int32),
            pltpu.VMEM((ch, DPAD), jnp.float32),
            pltpu.VMEM((ch, DPAD), jnp.float32),
            pltpu.SemaphoreType.DMA,
            pltpu.SemaphoreType.DMA,
        ],
    )
    def k(table_hbm, idx_hbm, out_hbm, idx_v, buf_a, buf_b, sem_a, sem_b):
        wid = lax.axis_index("s") * info.num_cores + lax.axis_index("c")

        pltpu.sync_copy(idx_hbm.at[wid], idx_v)
        pltpu.async_copy(table_hbm.at[idx_v.at[pl.ds(0, ch)]], buf_a, sem_a)

        def pair(h, carry):
            c0 = 2 * h
            i_b = idx_v.at[pl.ds((c0 + 1) * ch, ch)]
            pltpu.async_copy(table_hbm.at[i_b], buf_b, sem_b)
            i_a = idx_v.at[pl.ds(c0 * ch, ch)]
            pltpu.make_async_copy(table_hbm.at[i_a], buf_a, sem_a).wait()
            pltpu.sync_copy(buf_a, out_hbm.at[wid, c0])

            @pl.when(c0 + 2 < nchunk)
            def _():
                i_n = idx_v.at[pl.ds((c0 + 2) * ch, ch)]
                pltpu.async_copy(table_hbm.at[i_n], buf_a, sem_a)

            pltpu.make_async_copy(table_hbm.at[i_b], buf_b, sem_b).wait()
            pltpu.sync_copy(buf_b, out_hbm.at[wid, c0 + 1])
            return carry

        lax.fori_loop(0, nchunk // 2, pair, 0)
        if nchunk % 2:
            i_l = idx_v.at[pl.ds((nchunk - 1) * ch, ch)]
            pltpu.make_async_copy(table_hbm.at[i_l], buf_a, sem_a).wait()
            pltpu.sync_copy(buf_a, out_hbm.at[wid, nchunk - 1])

    out = k(table, idx2)
    return out.reshape(bidx, DPAD)


def _attn_body(tq_ref, nuv_ref, g_ref, wq_ref, wkgv_ref, wo_ref,
               ss_ref, out_ref):
    tq = tq_ref[...]
    blk = tq.shape[0]
    fq = tq[:, :E]
    G = g_ref[...]
    q = jnp.dot(fq, wq_ref[...], preferred_element_type=jnp.float32)
    zf = jnp.zeros((blk, E), jnp.float32)
    pb = jnp.zeros((blk, DPAD - E - 9), jnp.float32)
    qxa = jnp.concatenate([zf, tq[:, E:]], axis=1)
    qxb = jnp.concatenate([zf + 1.0, nuv_ref[...], pb], axis=1)
    g3 = G.reshape(blk, K, DPAD)
    x = ((g3 - qxa[:, None, :]) * qxb[:, None, :]).reshape(blk * K, DPAD)
    y = jnp.dot(x, wkgv_ref[...], preferred_element_type=jnp.float32)
    nk = y[:, :E]
    nv = y[:, E:]
    p = (nk.reshape(blk, K, E) * q[:, None, :]).reshape(blk * K, E)
    sr = jnp.dot(p, ss_ref[...], preferred_element_type=jnp.float32)
    e3 = jnp.exp(sr).reshape(blk, K, E)
    r = 1.0 / jnp.sum(e3, axis=1, keepdims=True)
    w3 = (e3 * r) * nv.reshape(blk, K, E)
    o = jnp.sum(w3, axis=1)
    nf = fq + jnp.dot(o, wo_ref[...], preferred_element_type=jnp.float32)
    out_ref[...] = jnp.concatenate([nf, tq[:, E:]], axis=1)


def _attention(table, nuv9, g2, wq, wkgv, wo, ss, m, tq_map,
               blk=400):
    nblk = m // blk
    assert nblk * blk == m
    return pl.pallas_call(
        _attn_body,
        grid=(nblk,),
        in_specs=[
            pl.BlockSpec((blk, DPAD), tq_map),
            pl.BlockSpec((blk, 9), tq_map),
            pl.BlockSpec((blk * K, DPAD), lambda i: (i, 0)),
            pl.BlockSpec((E, E), lambda i: (0, 0)),
            pl.BlockSpec((DPAD, 2 * E), lambda i: (0, 0)),
            pl.BlockSpec((E, E), lambda i: (0, 0)),
            pl.BlockSpec((E, E), lambda i: (0, 0)),
        ],
        out_specs=pl.BlockSpec((blk, DPAD), lambda i: (i, 0)),
        out_shape=jax.ShapeDtypeStruct((m, DPAD), jnp.float32),
    )(table, nuv9, g2, wq, wkgv, wo, ss)


def _pool_body(g1_ref, g2_ref, b_ref, w1t_ref, w2t_ref, w3t_ref, out_ref,
               acc_ref):
    i = pl.program_id(0)
    ng = pl.num_programs(0)

    @pl.when(i == 0)
    def _():
        acc_ref[...] = jnp.zeros_like(acc_ref)

    diff = g1_ref[:, :E] - g2_ref[:, :E]
    b = b_ref[...]
    blk = b.shape[0]
    oh = (b == lax.broadcasted_iota(jnp.int32, (blk, 16), 1)
          ).astype(jnp.float32)
    acc_ref[...] += lax.dot_general(diff, oh, (((0,), (0,)), ((), ())),
                                    precision=_HI,
                                    preferred_element_type=jnp.float32)

    @pl.when(i == ng - 1)
    def _():
        t = jnp.dot(w1t_ref[...], acc_ref[...],
                    preferred_element_type=jnp.float32)
        t = jnp.dot(w2t_ref[...], t, preferred_element_type=jnp.float32)
        out_ref[...] = jnp.dot(w3t_ref[...], t,
                               preferred_element_type=jnp.float32)


def _pool_head(g1, g2, batch2d, w1t, w2t, w3t):
    n = batch2d.shape[0]
    blk = 1000
    assert n % blk == 0
    nblk = n // blk
    out = pl.pallas_call(
        _pool_body,
        grid=(nblk,),
        in_specs=[
            pl.BlockSpec((blk, DPAD), lambda i: (i, 0)),
            pl.BlockSpec((blk, DPAD), lambda i: (i, 0)),
            pl.BlockSpec((blk, 1), lambda i: (i, 0)),
            pl.BlockSpec((E, E), lambda i: (0, 0)),
            pl.BlockSpec((E, E), lambda i: (0, 0)),
            pl.BlockSpec((1, E), lambda i: (0, 0)),
        ],
        out_specs=pl.BlockSpec((1, 16), lambda i: (0, 0)),
        out_shape=jax.ShapeDtypeStruct((1, 16), jnp.float32),
        scratch_shapes=[pltpu.VMEM((E, 16), jnp.float32)],
    )(g1, g2, batch2d, w1t, w2t, w3t)
    return out.reshape(16)


def kernel(token_p1, token_p2, token_p3, llm_p1, llm_p2, llm_p3, xyz_p1,
           xyz_p2, xyz_p3, nuv_p1, nuv_p2, nuv_p3, topk_p1, topk_p2, topk_p3,
           topk_i2, topk_i3, batch_p1, params):
    p = params
    n = llm_p1.shape[0]
    emb_pad = jnp.pad(p['emb_tok'].astype(jnp.float32), ((0, 11), (0, 0)))
    ln_g = p['ln_g'].reshape(1, -1)
    ln_b = p['ln_b'].reshape(1, -1)
    b1 = p['llm_b1'].reshape(1, -1)
    b2 = p['llm_b2'].reshape(1, -1)
    ss = jnp.kron(jnp.eye(H, dtype=jnp.float32),
                  jnp.ones((DH, DH), jnp.float32))
    scale = 1.0 / (DH ** 0.5)

    tabs = []
    for tok, llm, xyz in ((token_p1, llm_p1, xyz_p1),
                          (token_p2, llm_p2, xyz_p2),
                          (token_p3, llm_p3, xyz_p3)):
        tabs.append(_embed(tok.astype(jnp.int32).reshape(n, 1), llm, xyz,
                           emb_pad, ln_g, ln_b, p['llm_w1'], b1,
                           p['llm_w2'], b2))

    nuvs = [nuv_p1.reshape(n, 9), nuv_p2.reshape(n, 9), nuv_p3.reshape(n, 9)]
    idxs = [topk_p1.astype(jnp.int32).reshape(-1),
            topk_p2.astype(jnp.int32).reshape(-1),
            topk_p3.astype(jnp.int32).reshape(-1)]

    zkg = jnp.zeros((DPAD - E - 9, E), jnp.float32)
    zv = jnp.zeros((DPAD - E, E), jnp.float32)

    def wkgv_of(wk, wg, wv):
        wkg = jnp.concatenate([wk, jnp.repeat(wg, 3, axis=0), zkg], axis=0)
        wv2 = jnp.concatenate([wv, zv], axis=0)
        return jnp.concatenate([wkg, wv2], axis=1)

    ident = lambda i: (i, 0)
    for l in range(p['stru_Wq'].shape[0]):
        wq = p['stru_Wq'][l] * scale
        wkgv = wkgv_of(p['stru_Wk'][l], p['stru_Wg'][l], p['stru_Wv'][l])
        wo = p['stru_Wo'][l]
        gs = [_sc_gather(tabs[c], idxs[c]) for c in range(3)]
        tabs = [_attention(tabs[c], nuvs[c], gs[c], wq, wkgv, wo, ss,
                           n, ident) for c in range(3)]

    wq = p['inter_Wq'] * scale
    wkgv = wkgv_of(p['inter_Wk'], p['inter_Wg'], p['inter_Wv'])
    gi2 = _sc_gather(tabs[1], topk_i2.astype(jnp.int32).reshape(-1))
    gi3 = _sc_gather(tabs[2], topk_i3.astype(jnp.int32).reshape(-1))
    g1 = _attention(tabs[0], nuvs[0], gi2, wq, wkgv,
                    p['inter_Wo'], ss, n, ident)
    g2 = _attention(tabs[0], nuvs[0], gi3, wq, wkgv,
                    p['inter_Wo'], ss, n, ident)

    return _pool_head(g1, g2, batch_p1.astype(jnp.int32).reshape(n, 1),
                      p['out_w1'].T, p['out_w2'].T, p['out_w3'].T)


# speedup vs baseline: 1.0108x; 1.0108x over previous
import functools

import jax
import jax.numpy as jnp
from jax import lax
from jax.experimental import pallas as pl
from jax.experimental.pallas import tpu as pltpu
from jax.experimental.pallas import tpu_sc as plsc

K = 16
E = 64
H = 4
DH = E // H
DPAD = 128

_HI = jax.lax.Precision.HIGHEST


def _elu(x):
    return jnp.where(x > 0, x, jnp.exp(x) - 1.0)


def _embed_body(tok_ref, llm_ref, xyz_ref, emb_ref, g_ref, b_ref, w1_ref,
                b1_ref, w2_ref, b2_ref, out_ref):
    x = llm_ref[...]
    d = x.shape[1]
    m = jnp.sum(x, axis=1, keepdims=True) * (1.0 / d)
    v = jnp.sum(x * x, axis=1, keepdims=True) * (1.0 / d) - m * m
    h = (x - m) * (lax.rsqrt(v + 1e-5) * g_ref[...]) + b_ref[...]
    h = _elu(jnp.dot(h, w1_ref[...], preferred_element_type=jnp.float32)
             + b1_ref[...])
    h = _elu(jnp.dot(h, w2_ref[...], preferred_element_type=jnp.float32)
             + b2_ref[...])
    tok = tok_ref[...]
    blk = tok.shape[0]
    oh = (tok == lax.broadcasted_iota(jnp.int32, (blk, 32), 1)
          ).astype(jnp.float32)
    ft = jnp.dot(oh, emb_ref[...], precision=_HI,
                 preferred_element_type=jnp.float32)
    xyz = xyz_ref[...]
    pad = jnp.zeros((blk, DPAD - E - 9), jnp.float32)
    out_ref[...] = jnp.concatenate([ft, h, xyz, xyz, xyz, pad], axis=1)


def _embed(tok2d, llm, xyz, emb_pad, ln_g, ln_b, w1, b1, w2, b2):
    n, d = llm.shape
    blk = 1000
    assert n % blk == 0
    return pl.pallas_call(
        _embed_body,
        grid=(n // blk,),
        in_specs=[
            pl.BlockSpec((blk, 1), lambda i: (i, 0)),
            pl.BlockSpec((blk, d), lambda i: (i, 0)),
            pl.BlockSpec((blk, 3), lambda i: (i, 0)),
            pl.BlockSpec((32, 32), lambda i: (0, 0)),
            pl.BlockSpec((1, d), lambda i: (0, 0)),
            pl.BlockSpec((1, d), lambda i: (0, 0)),
            pl.BlockSpec((d, E), lambda i: (0, 0)),
            pl.BlockSpec((1, E), lambda i: (0, 0)),
            pl.BlockSpec((E, 32), lambda i: (0, 0)),
            pl.BlockSpec((1, 32), lambda i: (0, 0)),
        ],
        out_specs=pl.BlockSpec((blk, DPAD), lambda i: (i, 0)),
        out_shape=jax.ShapeDtypeStruct((n, DPAD), jnp.float32),
    )(tok2d, llm, xyz, emb_pad, ln_g, ln_b, w1, b1, w2, b2)


def _sc_gather(table, idx):
    bidx = idx.shape[0]
    info = plsc.get_sparse_core_info()
    nw = info.num_cores * info.num_subcores
    per_w = bidx // nw
    assert per_w * nw == bidx
    ch = 440
    while per_w % ch or ch % 8:
        ch -= 8
    nchunk = per_w // ch
    idx2 = idx.reshape(nw, per_w)
    mesh = plsc.VectorSubcoreMesh(core_axis_name="c", subcore_axis_name="s")

    @functools.partial(
        pl.kernel, mesh=mesh,
        out_type=jax.ShapeDtypeStruct((nw, nchunk, ch, DPAD), jnp.float32),
        scratch_types=[
            pltpu.VMEM((per_w,), jnp.int32),
            pltpu.VMEM((ch, DPAD), jnp.float32),
            pltpu.VMEM((ch, DPAD), jnp.float32),
            pltpu.SemaphoreType.DMA,
            pltpu.SemaphoreType.DMA,
        ],
    )
    def k(table_hbm, idx_hbm, out_hbm, idx_v, buf_a, buf_b, sem_a, sem_b):
        wid = lax.axis_index("s") * info.num_cores + lax.axis_index("c")

        pltpu.sync_copy(idx_hbm.at[wid], idx_v)
        pltpu.async_copy(table_hbm.at[idx_v.at[pl.ds(0, ch)]], buf_a, sem_a)

        def pair(h, carry):
            c0 = 2 * h
            i_b = idx_v.at[pl.ds((c0 + 1) * ch, ch)]
            pltpu.async_copy(table_hbm.at[i_b], buf_b, sem_b)
            i_a = idx_v.at[pl.ds(c0 * ch, ch)]
            pltpu.make_async_copy(table_hbm.at[i_a], buf_a, sem_a).wait()
            pltpu.sync_copy(buf_a, out_hbm.at[wid, c0])

            @pl.when(c0 + 2 < nchunk)
            def _():
                i_n = idx_v.at[pl.ds((c0 + 2) * ch, ch)]
                pltpu.async_copy(table_hbm.at[i_n], buf_a, sem_a)

            pltpu.make_async_copy(table_hbm.at[i_b], buf_b, sem_b).wait()
            pltpu.sync_copy(buf_b, out_hbm.at[wid, c0 + 1])
            return carry

        lax.fori_loop(0, nchunk // 2, pair, 0)
        if nchunk % 2:
            i_l = idx_v.at[pl.ds((nchunk - 1) * ch, ch)]
            pltpu.make_async_copy(table_hbm.at[i_l], buf_a, sem_a).wait()
            pltpu.sync_copy(buf_a, out_hbm.at[wid, nchunk - 1])

    out = k(table, idx2)
    return out.reshape(bidx, DPAD)


def _attn_body(tq_ref, nuv_ref, g_ref, wq_ref, wkgv_ref, wo_ref,
               ss_ref, out_ref):
    tq = tq_ref[...]
    blk = tq.shape[0]
    fq = tq[:, :E]
    G = g_ref[...]
    q = jnp.dot(fq, wq_ref[...], preferred_element_type=jnp.float32)
    zf = jnp.zeros((blk, E), jnp.float32)
    pb = jnp.zeros((blk, DPAD - E - 9), jnp.float32)
    qxa = jnp.concatenate([zf, tq[:, E:]], axis=1)
    qxb = jnp.concatenate([zf + 1.0, nuv_ref[...], pb], axis=1)
    g3 = G.reshape(blk, K, DPAD)
    x = ((g3 - qxa[:, None, :]) * qxb[:, None, :]).reshape(blk * K, DPAD)
    y = jnp.dot(x, wkgv_ref[...], preferred_element_type=jnp.float32)
    nk = y[:, :E]
    nv = y[:, E:]
    p = (nk.reshape(blk, K, E) * q[:, None, :]).reshape(blk * K, E)
    sr = jnp.dot(p, ss_ref[...], preferred_element_type=jnp.float32)
    e3 = jnp.exp(sr).reshape(blk, K, E)
    r = 1.0 / jnp.sum(e3, axis=1, keepdims=True)
    w3 = (e3 * r) * nv.reshape(blk, K, E)
    o = jnp.sum(w3, axis=1)
    nf = fq + jnp.dot(o, wo_ref[...], preferred_element_type=jnp.float32)
    out_ref[...] = jnp.concatenate([nf, tq[:, E:]], axis=1)


def _attention(table, nuv9, g2, wq, wkgv, wo, ss, m, tq_map,
               blk=1000):
    nblk = m // blk
    assert nblk * blk == m
    return pl.pallas_call(
        _attn_body,
        grid=(nblk,),
        in_specs=[
            pl.BlockSpec((blk, DPAD), tq_map),
            pl.BlockSpec((blk, 9), tq_map),
            pl.BlockSpec((blk * K, DPAD), lambda i: (i, 0)),
            pl.BlockSpec((E, E), lambda i: (0, 0)),
            pl.BlockSpec((DPAD, 2 * E), lambda i: (0, 0)),
            pl.BlockSpec((E, E), lambda i: (0, 0)),
            pl.BlockSpec((E, E), lambda i: (0, 0)),
        ],
        out_specs=pl.BlockSpec((blk, DPAD), lambda i: (i, 0)),
        out_shape=jax.ShapeDtypeStruct((m, DPAD), jnp.float32),
    )(table, nuv9, g2, wq, wkgv, wo, ss)


def _pool_body(g1_ref, g2_ref, b_ref, w1t_ref, w2t_ref, w3t_ref, out_ref,
               acc_ref):
    i = pl.program_id(0)
    ng = pl.num_programs(0)

    @pl.when(i == 0)
    def _():
        acc_ref[...] = jnp.zeros_like(acc_ref)

    diff = g1_ref[:, :E] - g2_ref[:, :E]
    b = b_ref[...]
    blk = b.shape[0]
    oh = (b == lax.broadcasted_iota(jnp.int32, (blk, 16), 1)
          ).astype(jnp.float32)
    acc_ref[...] += lax.dot_general(diff, oh, (((0,), (0,)), ((), ())),
                                    precision=_HI,
                                    preferred_element_type=jnp.float32)

    @pl.when(i == ng - 1)
    def _():
        t = jnp.dot(w1t_ref[...], acc_ref[...],
                    preferred_element_type=jnp.float32)
        t = jnp.dot(w2t_ref[...], t, preferred_element_type=jnp.float32)
        out_ref[...] = jnp.dot(w3t_ref[...], t,
                               preferred_element_type=jnp.float32)


def _pool_head(g1, g2, batch2d, w1t, w2t, w3t):
    n = batch2d.shape[0]
    blk = 1000
    assert n % blk == 0
    nblk = n // blk
    out = pl.pallas_call(
        _pool_body,
        grid=(nblk,),
        in_specs=[
            pl.BlockSpec((blk, DPAD), lambda i: (i, 0)),
            pl.BlockSpec((blk, DPAD), lambda i: (i, 0)),
            pl.BlockSpec((blk, 1), lambda i: (i, 0)),
            pl.BlockSpec((E, E), lambda i: (0, 0)),
            pl.BlockSpec((E, E), lambda i: (0, 0)),
            pl.BlockSpec((1, E), lambda i: (0, 0)),
        ],
        out_specs=pl.BlockSpec((1, 16), lambda i: (0, 0)),
        out_shape=jax.ShapeDtypeStruct((1, 16), jnp.float32),
        scratch_shapes=[pltpu.VMEM((E, 16), jnp.float32)],
    )(g1, g2, batch2d, w1t, w2t, w3t)
    return out.reshape(16)


def kernel(token_p1, token_p2, token_p3, llm_p1, llm_p2, llm_p3, xyz_p1,
           xyz_p2, xyz_p3, nuv_p1, nuv_p2, nuv_p3, topk_p1, topk_p2, topk_p3,
           topk_i2, topk_i3, batch_p1, params):
    p = params
    n = llm_p1.shape[0]
    emb_pad = jnp.pad(p['emb_tok'].astype(jnp.float32), ((0, 11), (0, 0)))
    ln_g = p['ln_g'].reshape(1, -1)
    ln_b = p['ln_b'].reshape(1, -1)
    b1 = p['llm_b1'].reshape(1, -1)
    b2 = p['llm_b2'].reshape(1, -1)
    ss = jnp.kron(jnp.eye(H, dtype=jnp.float32),
                  jnp.ones((DH, DH), jnp.float32))
    scale = 1.0 / (DH ** 0.5)

    tabs = []
    for tok, llm, xyz in ((token_p1, llm_p1, xyz_p1),
                          (token_p2, llm_p2, xyz_p2),
                          (token_p3, llm_p3, xyz_p3)):
        tabs.append(_embed(tok.astype(jnp.int32).reshape(n, 1), llm, xyz,
                           emb_pad, ln_g, ln_b, p['llm_w1'], b1,
                           p['llm_w2'], b2))

    nuvs = [nuv_p1.reshape(n, 9), nuv_p2.reshape(n, 9), nuv_p3.reshape(n, 9)]
    idxs = [topk_p1.astype(jnp.int32).reshape(-1),
            topk_p2.astype(jnp.int32).reshape(-1),
            topk_p3.astype(jnp.int32).reshape(-1)]

    zkg = jnp.zeros((DPAD - E - 9, E), jnp.float32)
    zv = jnp.zeros((DPAD - E, E), jnp.float32)

    def wkgv_of(wk, wg, wv):
        wkg = jnp.concatenate([wk, jnp.repeat(wg, 3, axis=0), zkg], axis=0)
        wv2 = jnp.concatenate([wv, zv], axis=0)
        return jnp.concatenate([wkg, wv2], axis=1)

    ident = lambda i: (i, 0)
    for l in range(p['stru_Wq'].shape[0]):
        wq = p['stru_Wq'][l] * scale
        wkgv = wkgv_of(p['stru_Wk'][l], p['stru_Wg'][l], p['stru_Wv'][l])
        wo = p['stru_Wo'][l]
        gs = [_sc_gather(tabs[c], idxs[c]) for c in range(3)]
        tabs = [_attention(tabs[c], nuvs[c], gs[c], wq, wkgv, wo, ss,
                           n, ident) for c in range(3)]

    wq = p['inter_Wq'] * scale
    wkgv = wkgv_of(p['inter_Wk'], p['inter_Wg'], p['inter_Wv'])
    gi2 = _sc_gather(tabs[1], topk_i2.astype(jnp.int32).reshape(-1))
    gi3 = _sc_gather(tabs[2], topk_i3.astype(jnp.int32).reshape(-1))
    g1 = _attention(tabs[0], nuvs[0], gi2, wq, wkgv,
                    p['inter_Wo'], ss, n, ident)
    g2 = _attention(tabs[0], nuvs[0], gi3, wq, wkgv,
                    p['inter_Wo'], ss, n, ident)

    return _pool_head(g1, g2, batch_p1.astype(jnp.int32).reshape(n, 1),
                      p['out_w1'].T, p['out_w2'].T, p['out_w3'].T)


# speedup vs baseline: 1.0196x; 1.0087x over previous
import functools

import jax
import jax.numpy as jnp
from jax import lax
from jax.experimental import pallas as pl
from jax.experimental.pallas import tpu as pltpu
from jax.experimental.pallas import tpu_sc as plsc

K = 16
E = 64
H = 4
DH = E // H
DPAD = 128

_HI = jax.lax.Precision.HIGHEST


def _elu(x):
    return jnp.where(x > 0, x, jnp.exp(x) - 1.0)


def _embed_body(tok_ref, llm_ref, xyz_ref, emb_ref, g_ref, b_ref, w1_ref,
                b1_ref, w2_ref, b2_ref, out_ref):
    x = llm_ref[...]
    d = x.shape[1]
    m = jnp.sum(x, axis=1, keepdims=True) * (1.0 / d)
    v = jnp.sum(x * x, axis=1, keepdims=True) * (1.0 / d) - m * m
    h = (x - m) * (lax.rsqrt(v + 1e-5) * g_ref[...]) + b_ref[...]
    h = _elu(jnp.dot(h, w1_ref[...], preferred_element_type=jnp.float32)
             + b1_ref[...])
    h = _elu(jnp.dot(h, w2_ref[...], preferred_element_type=jnp.float32)
             + b2_ref[...])
    tok = tok_ref[...]
    blk = tok.shape[0]
    oh = (tok == lax.broadcasted_iota(jnp.int32, (blk, 32), 1)
          ).astype(jnp.float32)
    ft = jnp.dot(oh, emb_ref[...], precision=_HI,
                 preferred_element_type=jnp.float32)
    xyz = xyz_ref[...]
    pad = jnp.zeros((blk, DPAD - E - 9), jnp.float32)
    out_ref[...] = jnp.concatenate([ft, h, xyz, xyz, xyz, pad], axis=1)


def _embed(tok2d, llm, xyz, emb_pad, ln_g, ln_b, w1, b1, w2, b2):
    n, d = llm.shape
    blk = 2000
    assert n % blk == 0
    return pl.pallas_call(
        _embed_body,
        grid=(n // blk,),
        in_specs=[
            pl.BlockSpec((blk, 1), lambda i: (i, 0)),
            pl.BlockSpec((blk, d), lambda i: (i, 0)),
            pl.BlockSpec((blk, 3), lambda i: (i, 0)),
            pl.BlockSpec((32, 32), lambda i: (0, 0)),
            pl.BlockSpec((1, d), lambda i: (0, 0)),
            pl.BlockSpec((1, d), lambda i: (0, 0)),
            pl.BlockSpec((d, E), lambda i: (0, 0)),
            pl.BlockSpec((1, E), lambda i: (0, 0)),
            pl.BlockSpec((E, 32), lambda i: (0, 0)),
            pl.BlockSpec((1, 32), lambda i: (0, 0)),
        ],
        out_specs=pl.BlockSpec((blk, DPAD), lambda i: (i, 0)),
        out_shape=jax.ShapeDtypeStruct((n, DPAD), jnp.float32),
    )(tok2d, llm, xyz, emb_pad, ln_g, ln_b, w1, b1, w2, b2)


def _sc_gather(table, idx):
    bidx = idx.shape[0]
    info = plsc.get_sparse_core_info()
    nw = info.num_cores * info.num_subcores
    per_w = bidx // nw
    assert per_w * nw == bidx
    ch = 440
    while per_w % ch or ch % 8:
        ch -= 8
    nchunk = per_w // ch
    idx2 = idx.reshape(nw, per_w)
    mesh = plsc.VectorSubcoreMesh(core_axis_name="c", subcore_axis_name="s")

    @functools.partial(
        pl.kernel, mesh=mesh,
        out_type=jax.ShapeDtypeStruct((nw, nchunk, ch, DPAD), jnp.float32),
        scratch_types=[
            pltpu.VMEM((per_w,), jnp.int32),
            pltpu.VMEM((ch, DPAD), jnp.float32),
            pltpu.VMEM((ch, DPAD), jnp.float32),
            pltpu.SemaphoreType.DMA,
            pltpu.SemaphoreType.DMA,
        ],
    )
    def k(table_hbm, idx_hbm, out_hbm, idx_v, buf_a, buf_b, sem_a, sem_b):
        wid = lax.axis_index("s") * info.num_cores + lax.axis_index("c")

        pltpu.sync_copy(idx_hbm.at[wid], idx_v)
        pltpu.async_copy(table_hbm.at[idx_v.at[pl.ds(0, ch)]], buf_a, sem_a)

        def pair(h, carry):
            c0 = 2 * h
            i_b = idx_v.at[pl.ds((c0 + 1) * ch, ch)]
            pltpu.async_copy(table_hbm.at[i_b], buf_b, sem_b)
            i_a = idx_v.at[pl.ds(c0 * ch, ch)]
            pltpu.make_async_copy(table_hbm.at[i_a], buf_a, sem_a).wait()
            pltpu.sync_copy(buf_a, out_hbm.at[wid, c0])

            @pl.when(c0 + 2 < nchunk)
            def _():
                i_n = idx_v.at[pl.ds((c0 + 2) * ch, ch)]
                pltpu.async_copy(table_hbm.at[i_n], buf_a, sem_a)

            pltpu.make_async_copy(table_hbm.at[i_b], buf_b, sem_b).wait()
            pltpu.sync_copy(buf_b, out_hbm.at[wid, c0 + 1])
            return carry

        lax.fori_loop(0, nchunk // 2, pair, 0)
        if nchunk % 2:
            i_l = idx_v.at[pl.ds((nchunk - 1) * ch, ch)]
            pltpu.make_async_copy(table_hbm.at[i_l], buf_a, sem_a).wait()
            pltpu.sync_copy(buf_a, out_hbm.at[wid, nchunk - 1])

    out = k(table, idx2)
    return out.reshape(bidx, DPAD)


def _attn_body(tq_ref, nuv_ref, g_ref, wq_ref, wkgv_ref, wo_ref,
               ss_ref, out_ref):
    tq = tq_ref[...]
    blk = tq.shape[0]
    fq = tq[:, :E]
    G = g_ref[...]
    q = jnp.dot(fq, wq_ref[...], preferred_element_type=jnp.float32)
    zf = jnp.zeros((blk, E), jnp.float32)
    pb = jnp.zeros((blk, DPAD - E - 9), jnp.float32)
    qxa = jnp.concatenate([zf, tq[:, E:]], axis=1)
    qxb = jnp.concatenate([zf + 1.0, nuv_ref[...], pb], axis=1)
    g3 = G.reshape(blk, K, DPAD)
    x = ((g3 - qxa[:, None, :]) * qxb[:, None, :]).reshape(blk * K, DPAD)
    y = jnp.dot(x, wkgv_ref[...], preferred_element_type=jnp.float32)
    nk = y[:, :E]
    nv = y[:, E:]
    p = (nk.reshape(blk, K, E) * q[:, None, :]).reshape(blk * K, E)
    sr = jnp.dot(p, ss_ref[...], preferred_element_type=jnp.float32)
    e3 = jnp.exp(sr).reshape(blk, K, E)
    r = 1.0 / jnp.sum(e3, axis=1, keepdims=True)
    w3 = (e3 * r) * nv.reshape(blk, K, E)
    o = jnp.sum(w3, axis=1)
    nf = fq + jnp.dot(o, wo_ref[...], preferred_element_type=jnp.float32)
    out_ref[...] = jnp.concatenate([nf, tq[:, E:]], axis=1)


def _attention(table, nuv9, g2, wq, wkgv, wo, ss, m, tq_map,
               blk=1000):
    nblk = m // blk
    assert nblk * blk == m
    return pl.pallas_call(
        _attn_body,
        grid=(nblk,),
        in_specs=[
            pl.BlockSpec((blk, DPAD), tq_map),
            pl.BlockSpec((blk, 9), tq_map),
            pl.BlockSpec((blk * K, DPAD), lambda i: (i, 0)),
            pl.BlockSpec((E, E), lambda i: (0, 0)),
            pl.BlockSpec((DPAD, 2 * E), lambda i: (0, 0)),
            pl.BlockSpec((E, E), lambda i: (0, 0)),
            pl.BlockSpec((E, E), lambda i: (0, 0)),
        ],
        out_specs=pl.BlockSpec((blk, DPAD), lambda i: (i, 0)),
        out_shape=jax.ShapeDtypeStruct((m, DPAD), jnp.float32),
    )(table, nuv9, g2, wq, wkgv, wo, ss)


def _pool_body(g1_ref, g2_ref, b_ref, w1t_ref, w2t_ref, w3t_ref, out_ref,
               acc_ref):
    i = pl.program_id(0)
    ng = pl.num_programs(0)

    @pl.when(i == 0)
    def _():
        acc_ref[...] = jnp.zeros_like(acc_ref)

    diff = g1_ref[:, :E] - g2_ref[:, :E]
    b = b_ref[...]
    blk = b.shape[0]
    oh = (b == lax.broadcasted_iota(jnp.int32, (blk, 16), 1)
          ).astype(jnp.float32)
    acc_ref[...] += lax.dot_general(diff, oh, (((0,), (0,)), ((), ())),
                                    precision=_HI,
                                    preferred_element_type=jnp.float32)

    @pl.when(i == ng - 1)
    def _():
        t = jnp.dot(w1t_ref[...], acc_ref[...],
                    preferred_element_type=jnp.float32)
        t = jnp.dot(w2t_ref[...], t, preferred_element_type=jnp.float32)
        out_ref[...] = jnp.dot(w3t_ref[...], t,
                               preferred_element_type=jnp.float32)


def _pool_head(g1, g2, batch2d, w1t, w2t, w3t):
    n = batch2d.shape[0]
    blk = 1000
    assert n % blk == 0
    nblk = n // blk
    out = pl.pallas_call(
        _pool_body,
        grid=(nblk,),
        in_specs=[
            pl.BlockSpec((blk, DPAD), lambda i: (i, 0)),
            pl.BlockSpec((blk, DPAD), lambda i: (i, 0)),
            pl.BlockSpec((blk, 1), lambda i: (i, 0)),
            pl.BlockSpec((E, E), lambda i: (0, 0)),
            pl.BlockSpec((E, E), lambda i: (0, 0)),
            pl.BlockSpec((1, E), lambda i: (0, 0)),
        ],
        out_specs=pl.BlockSpec((1, 16), lambda i: (0, 0)),
        out_shape=jax.ShapeDtypeStruct((1, 16), jnp.float32),
        scratch_shapes=[pltpu.VMEM((E, 16), jnp.float32)],
    )(g1, g2, batch2d, w1t, w2t, w3t)
    return out.reshape(16)


def kernel(token_p1, token_p2, token_p3, llm_p1, llm_p2, llm_p3, xyz_p1,
           xyz_p2, xyz_p3, nuv_p1, nuv_p2, nuv_p3, topk_p1, topk_p2, topk_p3,
           topk_i2, topk_i3, batch_p1, params):
    p = params
    n = llm_p1.shape[0]
    emb_pad = jnp.pad(p['emb_tok'].astype(jnp.float32), ((0, 11), (0, 0)))
    ln_g = p['ln_g'].reshape(1, -1)
    ln_b = p['ln_b'].reshape(1, -1)
    b1 = p['llm_b1'].reshape(1, -1)
    b2 = p['llm_b2'].reshape(1, -1)
    ss = jnp.kron(jnp.eye(H, dtype=jnp.float32),
                  jnp.ones((DH, DH), jnp.float32))
    scale = 1.0 / (DH ** 0.5)

    tabs = []
    for tok, llm, xyz in ((token_p1, llm_p1, xyz_p1),
                          (token_p2, llm_p2, xyz_p2),
                          (token_p3, llm_p3, xyz_p3)):
        tabs.append(_embed(tok.astype(jnp.int32).reshape(n, 1), llm, xyz,
                           emb_pad, ln_g, ln_b, p['llm_w1'], b1,
                           p['llm_w2'], b2))

    nuvs = [nuv_p1.reshape(n, 9), nuv_p2.reshape(n, 9), nuv_p3.reshape(n, 9)]
    idxs = [topk_p1.astype(jnp.int32).reshape(-1),
            topk_p2.astype(jnp.int32).reshape(-1),
            topk_p3.astype(jnp.int32).reshape(-1)]

    zkg = jnp.zeros((DPAD - E - 9, E), jnp.float32)
    zv = jnp.zeros((DPAD - E, E), jnp.float32)

    def wkgv_of(wk, wg, wv):
        wkg = jnp.concatenate([wk, jnp.repeat(wg, 3, axis=0), zkg], axis=0)
        wv2 = jnp.concatenate([wv, zv], axis=0)
        return jnp.concatenate([wkg, wv2], axis=1)

    ident = lambda i: (i, 0)
    for l in range(p['stru_Wq'].shape[0]):
        wq = p['stru_Wq'][l] * scale
        wkgv = wkgv_of(p['stru_Wk'][l], p['stru_Wg'][l], p['stru_Wv'][l])
        wo = p['stru_Wo'][l]
        gs = [_sc_gather(tabs[c], idxs[c]) for c in range(3)]
        tabs = [_attention(tabs[c], nuvs[c], gs[c], wq, wkgv, wo, ss,
                           n, ident) for c in range(3)]

    wq = p['inter_Wq'] * scale
    wkgv = wkgv_of(p['inter_Wk'], p['inter_Wg'], p['inter_Wv'])
    gi2 = _sc_gather(tabs[1], topk_i2.astype(jnp.int32).reshape(-1))
    gi3 = _sc_gather(tabs[2], topk_i3.astype(jnp.int32).reshape(-1))
    g1 = _attention(tabs[0], nuvs[0], gi2, wq, wkgv,
                    p['inter_Wo'], ss, n, ident)
    g2 = _attention(tabs[0], nuvs[0], gi3, wq, wkgv,
                    p['inter_Wo'], ss, n, ident)

    return _pool_head(g1, g2, batch_p1.astype(jnp.int32).reshape(n, 1),
                      p['out_w1'].T, p['out_w2'].T, p['out_w3'].T)
